# baseline (speedup 1.0000x reference)
import jax
import jax.numpy as jnp
from jax.experimental import pallas as pl

NB = 16
_BF = jnp.bfloat16


def _mm(a, b):
    return jax.lax.dot_general(a, b, (((1,), (0,)), ((), ())),
                               preferred_element_type=jnp.float32)


def _mm_hi(a, b):
    return jax.lax.dot_general(a, b, (((1,), (0,)), ((), ())),
                               precision=jax.lax.Precision.HIGHEST,
                               preferred_element_type=jnp.float32)


def _mm_rt(a, b):
    return jax.lax.dot_general(a, b, (((1,), (1,)), ((), ())),
                               precision=jax.lax.Precision.HIGHEST,
                               preferred_element_type=jnp.float32)


def _pad_h(x, lo, hi):
    cfg = ((0, 0, 0), (lo, hi, 0), (0, 0, 0), (0, 0, 0))
    return jax.lax.pad(x, jnp.asarray(0.0, x.dtype), cfg)


def _wshift_add(acc, y, ow):
    W, w = acc.shape[2], y.shape[2]
    cfg = ((0, 0, 0), (0, 0, 0), (ow, W - w - ow, 0), (0, 0, 0))
    return acc + jax.lax.pad(y, jnp.float32(0.0), cfg)


def _conv_gather(x4, wcat_ref, brow, K, pad):
    H, W = x4.shape[1], x4.shape[2]
    xb = x4.astype(_BF)
    xp = _pad_h(xb, pad, K - 1 - pad)
    p = jnp.concatenate([xp[:, kh:kh + H] for kh in range(K)], axis=-1)
    z = _mm(p.reshape(NB * H * W, K * 128), wcat_ref[...])
    acc = jnp.zeros((NB, H, W, 128), jnp.float32)
    for kw in range(K):
        zk = z[:, kw * 128:(kw + 1) * 128].reshape(NB, H, W, 128)
        dw_ = pad - kw
        os_w, oe_w = max(0, dw_), W + min(0, dw_)
        is_w, ie_w = max(0, -dw_), W + min(0, -dw_)
        acc = acc + jax.lax.pad(
            zk[:, :, is_w:ie_w, :], jnp.float32(0.0),
            ((0, 0, 0), (0, 0, 0), (os_w, W - (ie_w - is_w) - os_w, 0), (0, 0, 0)))
    return acc + brow


def _convt_s1(x4, dcat_ref, brow, K):
    Hin, Win = x4.shape[1], x4.shape[2]
    Ho, Wo = Hin + K - 1, Win + K - 1
    xb = x4.astype(_BF)
    xp = _pad_h(xb, K - 1, K - 1)
    p = jnp.concatenate([xp[:, K - 1 - kh:K - 1 - kh + Ho] for kh in range(K)],
                        axis=-1)
    z = _mm(p.reshape(NB * Ho * Win, K * 128), dcat_ref[...])
    acc = jnp.zeros((NB, Ho, Wo, 128), jnp.float32)
    for kw in range(K):
        zk = z[:, kw * 128:(kw + 1) * 128].reshape(NB, Ho, Win, 128)
        acc = _wshift_add(acc, zk, kw)
    return jnp.maximum(acc + brow, 0.0)


def _maxpool2(h, H):
    h5 = h.reshape(NB, H // 2, 2, H // 2, 2, 128)
    return jnp.max(jnp.max(h5, axis=4), axis=2)


def _vqvae_kernel(xcol_ref, xsq_ref, w1_ref, b1_ref, w2_ref, b2_ref,
                  w3_ref, b3_ref, d1_ref, db1_ref, d2_ref, db2_ref,
                  d3_ref, db3_ref, d4_ref, db4_ref, mw_ref, mb_ref,
                  dict_ref, idx_ref, rec_ref, m_ref):
    f32 = jnp.float32

    xcol = xcol_ref[...].reshape(NB * 576, 32)
    h1 = jnp.maximum(_mm(xcol, w1_ref[...]) + b1_ref[...], 0.0)

    b2 = b2_ref[...].reshape(1, 1, 1, 128)
    h2 = jnp.maximum(_conv_gather(h1.reshape(NB, 24, 24, 128), w2_ref, b2, 5, 2), 0.0)
    h2p = _maxpool2(h2, 24)

    b3 = b3_ref[...].reshape(1, 1, 1, 128)
    h3 = _conv_gather(h2p, w3_ref, b3, 5, 2)
    z = _maxpool2(h3, 12)
    zr = z.reshape(NB * 36, 128)

    dw = dict_ref[...]
    ones_row = jnp.ones((1, 128), f32)
    dn_row = _mm_rt(ones_row, dw * dw)
    scores = _mm_rt(zr, dw)
    t3 = (dn_row.reshape(1, 1, 128) - 2.0 * scores.reshape(NB, 36, 128))
    idx3 = jnp.argmax(t3, axis=-1).astype(jnp.int32)
    idx_ref[...] = idx3

    iota_k = jax.lax.broadcasted_iota(jnp.int32, (NB, 36, 128), 2)
    oh = (iota_k == idx3[:, :, None]).astype(f32).reshape(NB * 36, 128)
    val = _mm_hi(oh, dw)

    dv = val - zr
    m_ref[...] = jnp.sum(dv * dv, keepdims=True).reshape(1, 1, 1)

    db1 = db1_ref[...].reshape(1, 1, 1, 128)
    g1 = _convt_s1(val.reshape(NB, 6, 6, 128), d1_ref, db1, 4)

    g1b = g1.astype(_BF)
    g1p = _pad_h(g1b, 1, 1)
    sub = [[None, None], [None, None]]
    for ph in range(2):
        p = jnp.concatenate([g1p[:, 1 - a:11 - a] for a in range(2)], axis=-1)
        z2 = _mm(p.reshape(NB * 90, 256), d2_ref[:, ph * 512:(ph + 1) * 512])
        for pw in range(2):
            accp = jnp.zeros((NB, 10, 10, 128), jnp.float32)
            for b in range(2):
                zk = z2[:, (pw * 2 + b) * 128:(pw * 2 + b + 1) * 128]
                accp = _wshift_add(accp, zk.reshape(NB, 10, 9, 128), b)
            sub[ph][pw] = accp
    row0 = jnp.stack([sub[0][0], sub[0][1]], axis=3)
    row1 = jnp.stack([sub[1][0], sub[1][1]], axis=3)
    g2 = jnp.stack([row0, row1], axis=2).reshape(NB, 20, 20, 128)
    db2 = db2_ref[...].reshape(1, 1, 1, 128)
    g2 = jnp.maximum(g2 + db2, 0.0)

    db3 = db3_ref[...].reshape(1, 1, 1, 128)
    g3 = _convt_s1(g2, d3_ref, db3, 5)
    db4 = db4_ref[...].reshape(1, 1, 1, 128)
    g4 = _convt_s1(g3, d4_ref, db4, 5)

    g4b = g4.astype(_BF).astype(f32)
    mu = jnp.sum(g4b * mw_ref[...].reshape(1, 1, 1, 128), axis=-1) + mb_ref[...]
    diff = mu - xsq_ref[...]
    rec_ref[...] = jnp.sum(diff * diff, keepdims=True).reshape(1, 1, 1)


def kernel(x, ew1, eb1, ew2, eb2, ew3, eb3, dw1, db1, dw2, db2, dw3, db3,
           dw4, db4, mw, mb, dict_w):
    B = x.shape[0]
    G = B // NB
    f32 = jnp.float32

    xsq = x[:, 0]
    xcol = jnp.stack([xsq[:, kh:kh + 24, kw:kw + 24]
                      for kh in range(5) for kw in range(5)], axis=-1)
    xcol = jnp.pad(xcol, ((0, 0), (0, 0), (0, 0), (0, 7))).astype(_BF)
    w1m = jnp.pad(jnp.transpose(ew1.reshape(128, 25), (1, 0)),
                  ((0, 7), (0, 0))).astype(_BF)
    w2cat = jnp.transpose(ew2, (2, 1, 3, 0)).reshape(640, 640).astype(_BF)
    w3cat = jnp.transpose(ew3, (2, 1, 3, 0)).reshape(640, 640).astype(_BF)
    d1cat = jnp.transpose(dw1, (2, 0, 3, 1)).reshape(512, 512).astype(_BF)
    d3cat = jnp.transpose(dw3, (2, 0, 3, 1)).reshape(640, 640).astype(_BF)
    d4cat = jnp.transpose(dw4, (2, 0, 3, 1)).reshape(640, 640).astype(_BF)
    d2p = jnp.transpose(dw2, (2, 3, 0, 1)).reshape(2, 2, 2, 2, 128, 128)
    d2cat = jnp.transpose(d2p, (0, 4, 1, 3, 2, 5)).reshape(256, 1024).astype(_BF)
    mwrow = mw.reshape(1, 128)
    mbm = mb.reshape(1, 1)

    full = lambda *s: pl.BlockSpec(s, lambda i: (0,) * len(s))
    idx2, rec_p, m_p = pl.pallas_call(
        _vqvae_kernel,
        grid=(G,),
        in_specs=[
            pl.BlockSpec((NB, 24, 24, 32), lambda i: (i, 0, 0, 0)),
            pl.BlockSpec((NB, 28, 28), lambda i: (i, 0, 0)),
            full(32, 128), full(1, 128),
            full(640, 640), full(1, 128),
            full(640, 640), full(1, 128),
            full(512, 512), full(1, 128),
            full(256, 1024), full(1, 128),
            full(640, 640), full(1, 128),
            full(640, 640), full(1, 128),
            full(1, 128), full(1, 1),
            full(128, 128),
        ],
        out_specs=[
            pl.BlockSpec((NB, 36), lambda i: (i, 0)),
            pl.BlockSpec((1, 1, 1), lambda i: (i, 0, 0)),
            pl.BlockSpec((1, 1, 1), lambda i: (i, 0, 0)),
        ],
        out_shape=[
            jax.ShapeDtypeStruct((B, 36), jnp.int32),
            jax.ShapeDtypeStruct((G, 1, 1), f32),
            jax.ShapeDtypeStruct((G, 1, 1), f32),
        ],
    )(xcol, xsq, w1m, eb1[None], w2cat, eb2[None], w3cat, eb3[None],
      d1cat, db1[None], d2cat, db2[None], d3cat, db3[None], d4cat, db4[None],
      mwrow, mbm, dict_w)

    loss_rec = jnp.sum(rec_p) / (B * 784.0)
    m = jnp.sum(m_p) / (B * 36.0 * 128.0)
    dict_loss = m * 5.0
    enc_loss = m * 1.25
    var_loss = jnp.zeros((1,), f32)
    return (loss_rec, dict_loss, enc_loss, var_loss,
            idx2.reshape(B, 6, 6))


# speedup vs baseline: 1.1357x; 1.1357x over previous
import jax
import jax.numpy as jnp
from jax.experimental import pallas as pl

NB = 8
_BF = jnp.bfloat16


def _mm(a, b):
    return jax.lax.dot_general(a, b, (((1,), (0,)), ((), ())),
                               preferred_element_type=jnp.float32)


def _mm_hi(a, b):
    return jax.lax.dot_general(a, b, (((1,), (0,)), ((), ())),
                               precision=jax.lax.Precision.HIGHEST,
                               preferred_element_type=jnp.float32)


def _mm_rt(a, b):
    return jax.lax.dot_general(a, b, (((1,), (1,)), ((), ())),
                               precision=jax.lax.Precision.HIGHEST,
                               preferred_element_type=jnp.float32)


def _pad_h(x, lo, hi):
    cfg = ((0, 0, 0), (lo, hi, 0), (0, 0, 0), (0, 0, 0))
    return jax.lax.pad(x, jnp.asarray(0.0, x.dtype), cfg)


def _wshift_add(acc, y, ow):
    W, w = acc.shape[2], y.shape[2]
    cfg = ((0, 0, 0), (0, 0, 0), (ow, W - w - ow, 0), (0, 0, 0))
    return acc + jax.lax.pad(y, jnp.float32(0.0), cfg)


def _conv_gather(x4, wcat_ref, brow, K, pad):
    H, W = x4.shape[1], x4.shape[2]
    xp = _pad_h(x4, pad, K - 1 - pad)
    p = jnp.concatenate(
        [xp[:, kh:kh + H].reshape(NB * H * W, 128).astype(_BF)
         for kh in range(K)], axis=-1)
    z = _mm(p, wcat_ref[...])
    acc = jnp.zeros((NB, H, W, 128), jnp.float32)
    for kw in range(K):
        zk = z[:, kw * 128:(kw + 1) * 128].reshape(NB, H, W, 128)
        dw_ = pad - kw
        os_w, oe_w = max(0, dw_), W + min(0, dw_)
        is_w, ie_w = max(0, -dw_), W + min(0, -dw_)
        acc = acc + jax.lax.pad(
            zk[:, :, is_w:ie_w, :], jnp.float32(0.0),
            ((0, 0, 0), (0, 0, 0), (os_w, W - (ie_w - is_w) - os_w, 0), (0, 0, 0)))
    return acc + brow


def _convt_s1(x4, dcat_ref, brow, K):
    Hin, Win = x4.shape[1], x4.shape[2]
    Ho, Wo = Hin + K - 1, Win + K - 1
    xp = _pad_h(x4, K - 1, K - 1)
    p = jnp.concatenate(
        [xp[:, K - 1 - kh:K - 1 - kh + Ho].reshape(NB * Ho * Win, 128).astype(_BF)
         for kh in range(K)], axis=-1)
    z = _mm(p, dcat_ref[...])
    acc = jnp.zeros((NB, Ho, Wo, 128), jnp.float32)
    for kw in range(K):
        zk = z[:, kw * 128:(kw + 1) * 128].reshape(NB, Ho, Win, 128)
        acc = _wshift_add(acc, zk, kw)
    return jnp.maximum(acc + brow, 0.0)


def _maxpool2(h, H):
    h5 = h.reshape(NB, H // 2, 2, H // 2, 2, 128)
    return jnp.max(jnp.max(h5, axis=4), axis=2)


def _vqvae_kernel(xcol_ref, xsq_ref, w1_ref, b1_ref, w2_ref, b2_ref,
                  w3_ref, b3_ref, d1_ref, db1_ref, d2_ref, db2_ref,
                  d3_ref, db3_ref, d4_ref, db4_ref, mw_ref, mb_ref,
                  dict_ref, idx_ref, rec_ref, m_ref):
    f32 = jnp.float32

    xcol = xcol_ref[...].reshape(NB * 576, 32)
    h1 = jnp.maximum(_mm(xcol, w1_ref[...]) + b1_ref[...], 0.0)

    b2 = b2_ref[...].reshape(1, 1, 1, 128)
    h2 = jnp.maximum(_conv_gather(h1.reshape(NB, 24, 24, 128), w2_ref, b2, 5, 2), 0.0)
    h2p = _maxpool2(h2, 24)

    b3 = b3_ref[...].reshape(1, 1, 1, 128)
    h3 = _conv_gather(h2p, w3_ref, b3, 5, 2)
    z = _maxpool2(h3, 12)
    zr = z.reshape(NB * 36, 128)

    dw = dict_ref[...]
    ones_row = jnp.ones((1, 128), f32)
    dn_row = _mm_rt(ones_row, dw * dw)
    scores = _mm_rt(zr, dw)
    t3 = (dn_row.reshape(1, 1, 128) - 2.0 * scores.reshape(NB, 36, 128))
    idx3 = jnp.argmax(t3, axis=-1).astype(jnp.int32)
    idx_ref[...] = idx3

    iota_k = jax.lax.broadcasted_iota(jnp.int32, (NB, 36, 128), 2)
    oh = (iota_k == idx3[:, :, None]).astype(f32).reshape(NB * 36, 128)
    val = _mm_hi(oh, dw)

    dv = val - zr
    m_ref[...] = jnp.sum(dv * dv, keepdims=True).reshape(1, 1, 1)

    db1 = db1_ref[...].reshape(1, 1, 1, 128)
    g1 = _convt_s1(val.reshape(NB, 6, 6, 128), d1_ref, db1, 4)

    g1p = _pad_h(g1, 1, 1)
    sub = [[None, None], [None, None]]
    for ph in range(2):
        p = jnp.concatenate(
            [g1p[:, 1 - a:11 - a].reshape(NB * 90, 128).astype(_BF)
             for a in range(2)], axis=-1)
        z2 = _mm(p, d2_ref[:, ph * 512:(ph + 1) * 512])
        for pw in range(2):
            accp = jnp.zeros((NB, 10, 10, 128), jnp.float32)
            for b in range(2):
                zk = z2[:, (pw * 2 + b) * 128:(pw * 2 + b + 1) * 128]
                accp = _wshift_add(accp, zk.reshape(NB, 10, 9, 128), b)
            sub[ph][pw] = accp
    row0 = jnp.stack([sub[0][0], sub[0][1]], axis=3)
    row1 = jnp.stack([sub[1][0], sub[1][1]], axis=3)
    g2 = jnp.stack([row0, row1], axis=2).reshape(NB, 20, 20, 128)
    db2 = db2_ref[...].reshape(1, 1, 1, 128)
    g2 = jnp.maximum(g2 + db2, 0.0)

    db3 = db3_ref[...].reshape(1, 1, 1, 128)
    g3 = _convt_s1(g2, d3_ref, db3, 5)
    db4 = db4_ref[...].reshape(1, 1, 1, 128)
    g4 = _convt_s1(g3, d4_ref, db4, 5)

    g4b = g4.astype(_BF).astype(f32)
    mu = jnp.sum(g4b * mw_ref[...].reshape(1, 1, 1, 128), axis=-1) + mb_ref[...]
    diff = mu - xsq_ref[...]
    rec_ref[...] = jnp.sum(diff * diff, keepdims=True).reshape(1, 1, 1)


def kernel(x, ew1, eb1, ew2, eb2, ew3, eb3, dw1, db1, dw2, db2, dw3, db3,
           dw4, db4, mw, mb, dict_w):
    B = x.shape[0]
    G = B // NB
    f32 = jnp.float32

    xsq = x[:, 0]
    xcol = jnp.stack([xsq[:, kh:kh + 24, kw:kw + 24]
                      for kh in range(5) for kw in range(5)], axis=-1)
    xcol = jnp.pad(xcol, ((0, 0), (0, 0), (0, 0), (0, 7))).astype(_BF)
    w1m = jnp.pad(jnp.transpose(ew1.reshape(128, 25), (1, 0)),
                  ((0, 7), (0, 0))).astype(_BF)
    w2cat = jnp.transpose(ew2, (2, 1, 3, 0)).reshape(640, 640).astype(_BF)
    w3cat = jnp.transpose(ew3, (2, 1, 3, 0)).reshape(640, 640).astype(_BF)
    d1cat = jnp.transpose(dw1, (2, 0, 3, 1)).reshape(512, 512).astype(_BF)
    d3cat = jnp.transpose(dw3, (2, 0, 3, 1)).reshape(640, 640).astype(_BF)
    d4cat = jnp.transpose(dw4, (2, 0, 3, 1)).reshape(640, 640).astype(_BF)
    d2p = jnp.transpose(dw2, (2, 3, 0, 1)).reshape(2, 2, 2, 2, 128, 128)
    d2cat = jnp.transpose(d2p, (0, 4, 1, 3, 2, 5)).reshape(256, 1024).astype(_BF)
    mwrow = mw.reshape(1, 128)
    mbm = mb.reshape(1, 1)

    full = lambda *s: pl.BlockSpec(s, lambda i: (0,) * len(s))
    idx2, rec_p, m_p = pl.pallas_call(
        _vqvae_kernel,
        grid=(G,),
        in_specs=[
            pl.BlockSpec((NB, 24, 24, 32), lambda i: (i, 0, 0, 0)),
            pl.BlockSpec((NB, 28, 28), lambda i: (i, 0, 0)),
            full(32, 128), full(1, 128),
            full(640, 640), full(1, 128),
            full(640, 640), full(1, 128),
            full(512, 512), full(1, 128),
            full(256, 1024), full(1, 128),
            full(640, 640), full(1, 128),
            full(640, 640), full(1, 128),
            full(1, 128), full(1, 1),
            full(128, 128),
        ],
        out_specs=[
            pl.BlockSpec((NB, 36), lambda i: (i, 0)),
            pl.BlockSpec((1, 1, 1), lambda i: (i, 0, 0)),
            pl.BlockSpec((1, 1, 1), lambda i: (i, 0, 0)),
        ],
        out_shape=[
            jax.ShapeDtypeStruct((B, 36), jnp.int32),
            jax.ShapeDtypeStruct((G, 1, 1), f32),
            jax.ShapeDtypeStruct((G, 1, 1), f32),
        ],
    )(xcol, xsq, w1m, eb1[None], w2cat, eb2[None], w3cat, eb3[None],
      d1cat, db1[None], d2cat, db2[None], d3cat, db3[None], d4cat, db4[None],
      mwrow, mbm, dict_w)

    loss_rec = jnp.sum(rec_p) / (B * 784.0)
    m = jnp.sum(m_p) / (B * 36.0 * 128.0)
    dict_loss = m * 5.0
    enc_loss = m * 1.25
    var_loss = jnp.zeros((1,), f32)
    return (loss_rec, dict_loss, enc_loss, var_loss,
            idx2.reshape(B, 6, 6))
